# Initial kernel scaffold; baseline (speedup 1.0000x reference)
import jax
import jax.numpy as jnp
from jax.experimental import pallas as pl


def kernel(tokens, word_weight):
    raise NotImplementedError("write your pallas kernel here")


# speedup vs baseline: 9.1553x; 9.1553x over previous
import functools

import jax
import jax.numpy as jnp
from jax import lax
from jax.experimental import pallas as pl
from jax.experimental.pallas import tpu as pltpu
from jax.experimental.pallas import tpu_sc as plsc

L = 128
CG = 2
NBUF = 2


def kernel(tokens, word_weight):
    B, S = tokens.shape
    V, D = word_weight.shape
    N = B * S
    NC, NS = 2, 16
    NW = NC * NS
    GT = N // L
    per_w = GT // NW
    NCH = per_w // CG
    assert GT % NW == 0 and per_w % CG == 0 and NCH % NBUF == 0

    idx2 = tokens.reshape(GT, L).astype(jnp.int32)
    mesh = plsc.VectorSubcoreMesh(core_axis_name="c", subcore_axis_name="s")

    @functools.partial(
        pl.kernel,
        mesh=mesh,
        out_type=jax.ShapeDtypeStruct((GT, L, D), jnp.float32),
        scratch_types=[
            pltpu.VMEM((NBUF, CG, L), jnp.int32),
            pltpu.VMEM((NBUF, CG, L, D), jnp.float32),
            pltpu.SemaphoreType.DMA,
            pltpu.SemaphoreType.DMA,
            pltpu.SemaphoreType.DMA,
            pltpu.SemaphoreType.DMA,
        ],
    )
    def emb(table_hbm, idx_hbm, out_hbm, idx_v, rows_v, g0, g1, o0, o1):
        gsem = (g0, g1)
        osem = (o0, o1)
        wid = lax.axis_index("s") * NC + lax.axis_index("c")
        wbase = wid * per_w

        def fire(c, b):
            goff = wbase + c * CG
            pltpu.sync_copy(idx_hbm.at[pl.ds(goff, CG)], idx_v.at[b])
            for j in range(CG):
                pltpu.async_copy(
                    table_hbm.at[idx_v.at[b, j]], rows_v.at[b, j], gsem[b])

        def wait_gather(b):
            for j in range(CG):
                pltpu.make_async_copy(
                    table_hbm.at[idx_v.at[b, j]], rows_v.at[b, j],
                    gsem[b]).wait()

        def fire_writeout(c, b):
            goff = wbase + c * CG
            pltpu.async_copy(rows_v.at[b], out_hbm.at[pl.ds(goff, CG)],
                             osem[b])

        def wait_writeout(c, b):
            goff = wbase + c * CG
            pltpu.make_async_copy(rows_v.at[b], out_hbm.at[pl.ds(goff, CG)],
                                  osem[b]).wait()

        fire(0, 0)
        fire(1, 1)
        wait_gather(0)
        fire_writeout(0, 0)

        def pair(p, carry):
            for b in range(NBUF):
                v = p * NBUF + b
                wait_writeout(v - NBUF, b)
                fire(v, b)
                wait_gather(1 - b)
                fire_writeout(v - 1, 1 - b)
            return carry

        lax.fori_loop(1, NCH // NBUF, pair, 0)

        wait_gather(1)
        fire_writeout(NCH - 1, 1)
        wait_writeout(NCH - 2, 0)
        wait_writeout(NCH - 1, 1)

    out = emb(word_weight, idx2)
    return out.reshape(B, S, D)


# speedup vs baseline: 9.2616x; 1.0116x over previous
import functools

import jax
import jax.numpy as jnp
from jax import lax
from jax.experimental import pallas as pl
from jax.experimental.pallas import tpu as pltpu
from jax.experimental.pallas import tpu_sc as plsc

L = 128
CG = 2
NBUF = 3


def kernel(tokens, word_weight):
    B, S = tokens.shape
    V, D = word_weight.shape
    N = B * S
    NC, NS = 2, 16
    NW = NC * NS
    GT = N // L
    per_w = GT // NW
    NCH = per_w // CG
    assert GT % NW == 0 and per_w % CG == 0 and (NCH - 1) % NBUF == 0

    idx2 = tokens.reshape(GT, L).astype(jnp.int32)
    mesh = plsc.VectorSubcoreMesh(core_axis_name="c", subcore_axis_name="s")

    @functools.partial(
        pl.kernel,
        mesh=mesh,
        out_type=jax.ShapeDtypeStruct((GT, L, D), jnp.float32),
        scratch_types=[
            pltpu.VMEM((NBUF, CG, L), jnp.int32),
            pltpu.VMEM((NBUF, CG, L, D), jnp.float32),
            [pltpu.SemaphoreType.DMA] * NBUF,
            [pltpu.SemaphoreType.DMA] * NBUF,
            [pltpu.SemaphoreType.DMA] * NBUF,
        ],
    )
    def emb(table_hbm, idx_hbm, out_hbm, idx_v, rows_v, isem, gsem, osem):
        wid = lax.axis_index("s") * NC + lax.axis_index("c")
        wbase = wid * per_w

        def fire_idx(c, b):
            goff = wbase + c * CG
            pltpu.async_copy(idx_hbm.at[pl.ds(goff, CG)], idx_v.at[b],
                             isem[b])

        def wait_idx(b):
            pltpu.make_async_copy(idx_hbm.at[pl.ds(wbase, CG)], idx_v.at[b],
                                  isem[b]).wait()

        def fire_gather(b):
            for j in range(CG):
                pltpu.async_copy(
                    table_hbm.at[idx_v.at[b, j]], rows_v.at[b, j], gsem[b])

        def wait_gather(b):
            for j in range(CG):
                pltpu.make_async_copy(
                    table_hbm.at[idx_v.at[b, j]], rows_v.at[b, j],
                    gsem[b]).wait()

        def fire_writeout(c, b):
            goff = wbase + c * CG
            pltpu.async_copy(rows_v.at[b], out_hbm.at[pl.ds(goff, CG)],
                             osem[b])

        def wait_writeout(b):
            pltpu.make_async_copy(rows_v.at[b], out_hbm.at[pl.ds(wbase, CG)],
                                  osem[b]).wait()

        for c in range(NBUF):
            fire_idx(c, c)
        wait_idx(0)
        fire_gather(0)
        wait_idx(1)
        fire_gather(1)
        wait_gather(0)
        fire_writeout(0, 0)
        fire_idx(NBUF, 0)
        wait_idx(2)
        fire_gather(2)
        wait_gather(1)
        fire_writeout(1, 1)
        fire_idx(NBUF + 1, 1)

        def trio(p, carry):
            for b in range(NBUF):
                v = p * NBUF + b
                wait_gather((b - 1) % NBUF)
                fire_writeout(v - 1, (b - 1) % NBUF)

                @pl.when(v + 2 < NCH)
                def _():
                    fire_idx(v + 2, (b + 2) % NBUF)

                wait_writeout(b)
                wait_idx(b)
                fire_gather(b)
            return carry

        lax.fori_loop(1, (NCH - 1) // NBUF, trio, 0)

        bl = (NCH - 1) % NBUF
        wait_gather((bl - 1) % NBUF)
        fire_writeout(NCH - 2, (bl - 1) % NBUF)
        wait_writeout(bl)
        wait_idx(bl)
        fire_gather(bl)
        wait_gather(bl)
        fire_writeout(NCH - 1, bl)
        for b in range(NBUF):
            wait_writeout(b)

    out = emb(word_weight, idx2)
    return out.reshape(B, S, D)


# speedup vs baseline: 9.2872x; 1.0028x over previous
import functools

import jax
import jax.numpy as jnp
from jax import lax
from jax.experimental import pallas as pl
from jax.experimental.pallas import tpu as pltpu
from jax.experimental.pallas import tpu_sc as plsc

L = 256
CG = 1
NBUF = 3


def kernel(tokens, word_weight):
    B, S = tokens.shape
    V, D = word_weight.shape
    N = B * S
    NC, NS = 2, 16
    NW = NC * NS
    GT = N // L
    per_w = GT // NW
    NCH = per_w // CG
    assert GT % NW == 0 and per_w % CG == 0 and (NCH - 1) % NBUF == 0

    idx2 = tokens.reshape(GT, L).astype(jnp.int32)
    mesh = plsc.VectorSubcoreMesh(core_axis_name="c", subcore_axis_name="s")

    @functools.partial(
        pl.kernel,
        mesh=mesh,
        out_type=jax.ShapeDtypeStruct((GT, L, D), jnp.float32),
        scratch_types=[
            pltpu.VMEM((NBUF, CG, L), jnp.int32),
            pltpu.VMEM((NBUF, CG, L, D), jnp.float32),
            [pltpu.SemaphoreType.DMA] * NBUF,
            [pltpu.SemaphoreType.DMA] * NBUF,
            [pltpu.SemaphoreType.DMA] * NBUF,
        ],
    )
    def emb(table_hbm, idx_hbm, out_hbm, idx_v, rows_v, isem, gsem, osem):
        wid = lax.axis_index("s") * NC + lax.axis_index("c")
        wbase = wid * per_w

        def fire_idx(c, b):
            goff = wbase + c * CG
            pltpu.async_copy(idx_hbm.at[pl.ds(goff, CG)], idx_v.at[b],
                             isem[b])

        def wait_idx(b):
            pltpu.make_async_copy(idx_hbm.at[pl.ds(wbase, CG)], idx_v.at[b],
                                  isem[b]).wait()

        def fire_gather(b):
            for j in range(CG):
                pltpu.async_copy(
                    table_hbm.at[idx_v.at[b, j]], rows_v.at[b, j], gsem[b])

        def wait_gather(b):
            for j in range(CG):
                pltpu.make_async_copy(
                    table_hbm.at[idx_v.at[b, j]], rows_v.at[b, j],
                    gsem[b]).wait()

        def fire_writeout(c, b):
            goff = wbase + c * CG
            pltpu.async_copy(rows_v.at[b], out_hbm.at[pl.ds(goff, CG)],
                             osem[b])

        def wait_writeout(b):
            pltpu.make_async_copy(rows_v.at[b], out_hbm.at[pl.ds(wbase, CG)],
                                  osem[b]).wait()

        for c in range(NBUF):
            fire_idx(c, c)
        wait_idx(0)
        fire_gather(0)
        wait_idx(1)
        fire_gather(1)
        wait_gather(0)
        fire_writeout(0, 0)
        fire_idx(NBUF, 0)
        wait_idx(2)
        fire_gather(2)
        wait_gather(1)
        fire_writeout(1, 1)
        fire_idx(NBUF + 1, 1)

        def trio(p, carry):
            for b in range(NBUF):
                v = p * NBUF + b
                wait_gather((b - 1) % NBUF)
                fire_writeout(v - 1, (b - 1) % NBUF)

                @pl.when(v + 2 < NCH)
                def _():
                    fire_idx(v + 2, (b + 2) % NBUF)

                wait_writeout(b)
                wait_idx(b)
                fire_gather(b)
            return carry

        lax.fori_loop(1, (NCH - 1) // NBUF, trio, 0)

        bl = (NCH - 1) % NBUF
        wait_gather((bl - 1) % NBUF)
        fire_writeout(NCH - 2, (bl - 1) % NBUF)
        wait_writeout(bl)
        wait_idx(bl)
        fire_gather(bl)
        wait_gather(bl)
        fire_writeout(NCH - 1, bl)
        for b in range(NBUF):
            wait_writeout(b)

    out = emb(word_weight, idx2)
    return out.reshape(B, S, D)
